# baseline (speedup 1.0000x reference)
import jax
import jax.numpy as jnp
from jax.experimental import pallas as pl


def _copy2_body(src_ref, src2_ref, dst_ref, dst2_ref):
    dst_ref[...] = src_ref[...]
    dst2_ref[...] = src2_ref[...]


def kernel(output, bank):
    dim, size = bank.shape
    b, d = output.shape
    blk = 8192
    grid = size // blk
    rblk = b // grid
    snap, out_copy = pl.pallas_call(
        _copy2_body,
        grid=(grid,),
        in_specs=[
            pl.BlockSpec((dim, blk), lambda i: (0, i)),
            pl.BlockSpec((rblk, d), lambda i: (i, 0)),
        ],
        out_specs=[
            pl.BlockSpec((dim, blk), lambda i: (0, i)),
            pl.BlockSpec((rblk, d), lambda i: (i, 0)),
        ],
        out_shape=[
            jax.ShapeDtypeStruct(bank.shape, bank.dtype),
            jax.ShapeDtypeStruct(output.shape, output.dtype),
        ],
    )(bank, output)
    return (out_copy, snap)


# speedup vs baseline: 1.0185x; 1.0185x over previous
import jax
import jax.numpy as jnp
from jax.experimental import pallas as pl


def _copy2_body(src_ref, src2_ref, dst_ref, dst2_ref):
    dst_ref[...] = src_ref[...]
    dst2_ref[...] = src2_ref[...]


def kernel(output, bank):
    dim, size = bank.shape
    b, d = output.shape
    blk = 16384
    grid = size // blk
    rblk = b // grid
    snap, out_copy = pl.pallas_call(
        _copy2_body,
        grid=(grid,),
        in_specs=[
            pl.BlockSpec((dim, blk), lambda i: (0, i)),
            pl.BlockSpec((rblk, d), lambda i: (i, 0)),
        ],
        out_specs=[
            pl.BlockSpec((dim, blk), lambda i: (0, i)),
            pl.BlockSpec((rblk, d), lambda i: (i, 0)),
        ],
        out_shape=[
            jax.ShapeDtypeStruct(bank.shape, bank.dtype),
            jax.ShapeDtypeStruct(output.shape, output.dtype),
        ],
    )(bank, output)
    return (out_copy, snap)


# speedup vs baseline: 1.0198x; 1.0012x over previous
import jax
from jax.experimental import pallas as pl


def _copy2_body(src_ref, src2_ref, dst_ref, dst2_ref):
    dst_ref[...] = src_ref[...]
    dst2_ref[...] = src2_ref[...]


def kernel(output, bank):
    dim, size = bank.shape
    b, d = output.shape
    blk = 16384
    grid = size // blk
    rblk = b // grid
    snap, out_copy = pl.pallas_call(
        _copy2_body,
        grid=(grid,),
        in_specs=[
            pl.BlockSpec((dim, blk), lambda i: (0, i)),
            pl.BlockSpec((rblk, d), lambda i: (i, 0)),
        ],
        out_specs=[
            pl.BlockSpec((dim, blk), lambda i: (0, i)),
            pl.BlockSpec((rblk, d), lambda i: (i, 0)),
        ],
        out_shape=[
            jax.ShapeDtypeStruct(bank.shape, bank.dtype),
            jax.ShapeDtypeStruct(output.shape, output.dtype),
        ],
    )(bank, output)
    return (out_copy, snap)
